# baseline (speedup 1.0000x reference)
import functools

import jax
import jax.numpy as jnp
from jax import lax
from jax.experimental import pallas as pl
from jax.experimental.pallas import tpu as pltpu
from jax.experimental.pallas import tpu_sc as plsc

D_MODEL = 64
SCALE = 8.0
NUM_CORES = 2
NUM_SUBCORES = 16
NUM_WORKERS = NUM_CORES * NUM_SUBCORES
BLK = 128
LANES = 16


@functools.lru_cache(maxsize=None)
def _make_kernel(seq: int, batch: int, vocab_pairs: int):
    n_blocks = seq
    mesh = plsc.VectorSubcoreMesh(core_axis_name="c", subcore_axis_name="s")

    @functools.partial(
        pl.kernel,
        mesh=mesh,
        out_type=jax.ShapeDtypeStruct((seq, D_MODEL, batch), jnp.float32),
        scratch_types=[
            pltpu.VMEM((seq, BLK), jnp.int32),
            pltpu.VMEM((2, BLK), jnp.int32),
            pltpu.VMEM((2, BLK), jnp.int32),
            pltpu.VMEM((2, BLK, 2 * D_MODEL), jnp.float32),
            pltpu.VMEM((2, D_MODEL, BLK), jnp.float32),
            pltpu.SemaphoreType.DMA,
            pltpu.SemaphoreType.DMA,
            pltpu.SemaphoreType.DMA,
            pltpu.SemaphoreType.DMA,
        ],
        compiler_params=pltpu.CompilerParams(needs_layout_passes=False),
    )
    def emb(xt_hbm, tab2_hbm, out_hbm, idx_v, ridx_v, hb_v, gbuf, obuf,
            gsem0, gsem1, wsem0, wsem1):
        wid = lax.axis_index("s") * NUM_CORES + lax.axis_index("c")
        col0 = wid * BLK
        pltpu.sync_copy(xt_hbm.at[:, pl.ds(col0, BLK)], idx_v)

        gsems = (gsem0, gsem1)
        wsems = (wsem0, wsem1)

        def prep(slot, g):
            for q in range(BLK // LANES):
                tok = idx_v[g, pl.ds(LANES * q, LANES)]
                ridx_v[slot, pl.ds(LANES * q, LANES)] = tok >> 1
                hb_v[slot, pl.ds(LANES * q, LANES)] = (tok & 1) << 6

        def fire(slot):
            return pltpu.async_copy(
                tab2_hbm.at[ridx_v.at[slot]], gbuf.at[slot], gsems[slot])

        def wait_gather(slot):
            pltpu.make_async_copy(
                tab2_hbm.at[ridx_v.at[slot]], gbuf.at[slot], gsems[slot]
            ).wait()

        def wait_write(slot):
            pltpu.make_async_copy(
                obuf.at[slot], out_hbm.at[0, :, pl.ds(col0, BLK)], wsems[slot]
            ).wait()

        rvecs = [lax.iota(jnp.int32, LANES) + LANES * q
                 for q in range(BLK // LANES)]

        def emit(slot, g, s):
            @pl.when(s >= 1)
            def _():
                wait_write(slot)

            hbs = [hb_v[slot, pl.ds(LANES * q, LANES)]
                   for q in range(BLK // LANES)]

            def cbody(c, _):
                for q in range(BLK // LANES):
                    vec = plsc.load_gather(gbuf.at[slot], [rvecs[q], hbs[q] + c])
                    obuf[slot, c, pl.ds(LANES * q, LANES)] = vec * SCALE
                return 0

            lax.fori_loop(0, D_MODEL, cbody, 0)
            pltpu.async_copy(
                obuf.at[slot], out_hbm.at[g, :, pl.ds(col0, BLK)], wsems[slot])

        prep(0, 0)
        fire(0)

        def body(s, _):
            g0 = 2 * s
            g1 = g0 + 1
            prep(1, g1)
            fire(1)
            wait_gather(0)
            emit(0, g0, s)

            @pl.when(s < n_blocks // 2 - 1)
            def _():
                prep(0, g0 + 2)
                fire(0)

            wait_gather(1)
            emit(1, g1, s)
            return 0

        lax.fori_loop(0, n_blocks // 2, body, 0)
        wait_write(0)
        wait_write(1)

    return emb


@jax.jit
def kernel(x, table):
    batch, seq = x.shape
    vocab = table.shape[0]
    xt = x.T.astype(jnp.int32)
    tab2 = table.reshape(vocab // 2, 2 * D_MODEL)
    out_t = _make_kernel(seq, batch, vocab // 2)(xt, tab2)
    return jnp.transpose(out_t, (2, 0, 1))


# speedup vs baseline: 1.1595x; 1.1595x over previous
import functools

import jax
import jax.numpy as jnp
from jax import lax
from jax.experimental import pallas as pl
from jax.experimental.pallas import tpu as pltpu
from jax.experimental.pallas import tpu_sc as plsc

D_MODEL = 64
SCALE = 8.0
NUM_CORES = 2
NUM_SUBCORES = 16
NUM_WORKERS = NUM_CORES * NUM_SUBCORES
BLK = 128
ROW = 2 * D_MODEL
LANES = 16
PITCH = 129


@functools.lru_cache(maxsize=None)
def _make_kernel(seq: int, batch: int):
    n_blocks = seq
    mesh = plsc.VectorSubcoreMesh(core_axis_name="c", subcore_axis_name="s")

    @functools.partial(
        pl.kernel,
        mesh=mesh,
        out_type=jax.ShapeDtypeStruct((seq, D_MODEL, batch), jnp.float32),
        scratch_types=[
            pltpu.VMEM((seq, BLK), jnp.int32),
            pltpu.VMEM((2, BLK, ROW), jnp.float32),
            pltpu.VMEM((2, D_MODEL, PITCH), jnp.float32),
            pltpu.SemaphoreType.DMA,
            pltpu.SemaphoreType.DMA,
            pltpu.SemaphoreType.DMA,
            pltpu.SemaphoreType.DMA,
        ],
        compiler_params=pltpu.CompilerParams(needs_layout_passes=False),
    )
    def emb(xt_hbm, tabp_hbm, out_hbm, idx_v, gbuf, obuf,
            gsem0, gsem1, wsem0, wsem1):
        wid = lax.axis_index("s") * NUM_CORES + lax.axis_index("c")
        col0 = wid * BLK
        pltpu.sync_copy(xt_hbm.at[:, pl.ds(col0, BLK)], idx_v)

        gsems = (gsem0, gsem1)
        wsems = (wsem0, wsem1)

        def fire(slot, g):
            pltpu.async_copy(
                tabp_hbm.at[idx_v.at[g]], gbuf.at[slot], gsems[slot])

        def wait_gather(slot, g):
            pltpu.make_async_copy(
                tabp_hbm.at[idx_v.at[g]], gbuf.at[slot], gsems[slot]
            ).wait()

        def wait_write(slot):
            pltpu.make_async_copy(
                obuf.at[slot, :, pl.ds(0, BLK)],
                out_hbm.at[0, :, pl.ds(col0, BLK)], wsems[slot]
            ).wait()

        rowq = [lax.iota(jnp.int32, LANES) + LANES * q
                for q in range(D_MODEL // LANES)]

        def emit(slot, g, s):
            @pl.when(s >= 1)
            def _():
                wait_write(slot)

            def jbody(j, _):
                jvec = jnp.zeros((LANES,), jnp.int32) + j
                for q in range(D_MODEL // LANES):
                    vec = gbuf[slot, j, pl.ds(LANES * q, LANES)] * SCALE
                    plsc.store_scatter(obuf.at[slot], [rowq[q], jvec], vec)
                return 0

            lax.fori_loop(0, BLK, jbody, 0)
            pltpu.async_copy(
                obuf.at[slot, :, pl.ds(0, BLK)],
                out_hbm.at[g, :, pl.ds(col0, BLK)], wsems[slot])

        fire(0, 0)

        def body(s, _):
            g0 = 2 * s
            g1 = g0 + 1
            fire(1, g1)
            wait_gather(0, g0)
            emit(0, g0, s)

            @pl.when(s < n_blocks // 2 - 1)
            def _():
                fire(0, g0 + 2)

            wait_gather(1, g1)
            emit(1, g1, s)
            return 0

        lax.fori_loop(0, n_blocks // 2, body, 0)
        wait_write(0)
        wait_write(1)

    return emb


@jax.jit
def kernel(x, table):
    batch, seq = x.shape
    xt = x.T.astype(jnp.int32)
    tabp = jnp.pad(table, ((0, 0), (0, ROW - D_MODEL)))
    out_t = _make_kernel(seq, batch)(xt, tabp)
    return jnp.transpose(out_t, (2, 0, 1))


# speedup vs baseline: 2.5345x; 2.1859x over previous
import functools

import jax
import jax.numpy as jnp
from jax import lax
from jax.experimental import pallas as pl
from jax.experimental.pallas import tpu as pltpu
from jax.experimental.pallas import tpu_sc as plsc

D_MODEL = 64
SCALE = 8.0
NUM_CORES = 2
NUM_SUBCORES = 16
NUM_WORKERS = NUM_CORES * NUM_SUBCORES
BLK = 128
ROW = 2 * D_MODEL
LANES = 16
PITCH = 129


@functools.lru_cache(maxsize=None)
def _make_kernel(seq: int, batch: int):
    n_blocks = seq
    mesh = plsc.VectorSubcoreMesh(core_axis_name="c", subcore_axis_name="s")

    @functools.partial(
        pl.kernel,
        mesh=mesh,
        out_type=jax.ShapeDtypeStruct((seq, D_MODEL, batch), jnp.float32),
        scratch_types=[
            pltpu.VMEM((seq, BLK), jnp.int32),
            pltpu.VMEM((2, BLK, ROW), jnp.float32),
            pltpu.VMEM((2, D_MODEL, PITCH), jnp.float32),
            pltpu.SemaphoreType.DMA,
            pltpu.SemaphoreType.DMA,
            pltpu.SemaphoreType.DMA,
            pltpu.SemaphoreType.DMA,
        ],
        compiler_params=pltpu.CompilerParams(needs_layout_passes=False),
    )
    def emb(xt_hbm, tabp_hbm, out_hbm, idx_v, gbuf, obuf,
            gsem0, gsem1, wsem0, wsem1):
        wid = lax.axis_index("s") * NUM_CORES + lax.axis_index("c")
        col0 = wid * BLK
        pltpu.sync_copy(xt_hbm.at[:, pl.ds(col0, BLK)], idx_v)

        gsems = (gsem0, gsem1)
        wsems = (wsem0, wsem1)

        def fire(slot, g):
            pltpu.async_copy(
                tabp_hbm.at[idx_v.at[g]], gbuf.at[slot], gsems[slot])

        def wait_gather(slot, g):
            pltpu.make_async_copy(
                tabp_hbm.at[idx_v.at[g]], gbuf.at[slot], gsems[slot]
            ).wait()

        def wait_write(slot):
            pltpu.make_async_copy(
                obuf.at[slot, :, pl.ds(0, BLK)],
                out_hbm.at[0, :, pl.ds(col0, BLK)], wsems[slot]
            ).wait()

        rowq = [lax.iota(jnp.int32, LANES) + LANES * q
                for q in range(D_MODEL // LANES)]

        def emit(slot, g, s):
            @pl.when(s >= 1)
            def _():
                wait_write(slot)

            @functools.partial(plsc.parallel_loop, 0, BLK, unroll=8)
            def jbody(j):
                jvec = jnp.zeros((LANES,), jnp.int32) + j
                for q in range(D_MODEL // LANES):
                    vec = gbuf[slot, j, pl.ds(LANES * q, LANES)] * SCALE
                    plsc.store_scatter(obuf.at[slot], [rowq[q], jvec], vec)
            pltpu.async_copy(
                obuf.at[slot, :, pl.ds(0, BLK)],
                out_hbm.at[g, :, pl.ds(col0, BLK)], wsems[slot])

        fire(0, 0)

        def body(s, _):
            g0 = 2 * s
            g1 = g0 + 1
            fire(1, g1)
            wait_gather(0, g0)
            emit(0, g0, s)

            @pl.when(s < n_blocks // 2 - 1)
            def _():
                fire(0, g0 + 2)

            wait_gather(1, g1)
            emit(1, g1, s)
            return 0

        lax.fori_loop(0, n_blocks // 2, body, 0)
        wait_write(0)
        wait_write(1)

    return emb


@jax.jit
def kernel(x, table):
    batch, seq = x.shape
    xt = x.T.astype(jnp.int32)
    tabp = jnp.pad(table, ((0, 0), (0, ROW - D_MODEL)))
    out_t = _make_kernel(seq, batch)(xt, tabp)
    return jnp.transpose(out_t, (2, 0, 1))
